# baseline (speedup 1.0000x reference)
import jax
import jax.numpy as jnp
from jax import lax
from jax.experimental import pallas as pl
from jax.experimental.pallas import tpu as pltpu

_C = 96
_W = 224
_BR = 112


def _build_placed(cm_col):
    cc = lax.broadcasted_iota(jnp.int32, (_C, _C), 0)
    kk = lax.broadcasted_iota(jnp.int32, (_C, _C), 1)
    tri_le = (kk <= cc).astype(jnp.float32)
    rank_inc = jnp.dot(tri_le, cm_col,
                       preferred_element_type=jnp.float32)
    total_kept = jnp.sum(cm_col)
    rank = rank_inc - 1.0
    kkf = kk.astype(jnp.float32)
    placed = jnp.where(rank == kkf, 1.0, 0.0) * cm_col
    pad = jnp.where((cc == 0) & (kkf >= total_kept), 1.0, 0.0)
    return placed + pad


def _body(xi_ref, xo_ref, o_ref, acc_ref, placed_ref):
    p = pl.program_id(0)
    b = pl.program_id(1)
    t = pl.program_id(2)
    r = pl.program_id(3)

    @pl.when((p == 0) & (b == 0) & (t == 0) & (r == 0))
    def _init():
        acc_ref[...] = jnp.zeros_like(acc_ref)

    @pl.when(p == 0)
    def _mask():
        nz = (xi_ref[...] != 0.0).astype(jnp.float32)
        acc_ref[...] = jnp.maximum(acc_ref[...], jnp.max(nz, axis=(0, 1)))

    @pl.when((p == 0) & (b == 3) & (t == 1) & (r == 1))
    def _finalize():
        cm_col = jnp.max(acc_ref[...], axis=1, keepdims=True)
        placed_ref[...] = _build_placed(cm_col)

    @pl.when(p == 1)
    def _gather():
        placed = placed_ref[...]
        for i in range(_BR):
            o_ref[0, 0, i] = lax.dot_general(
                placed, xo_ref[0, 0, i],
                dimension_numbers=(((0,), (0,)), ((), ())),
                preferred_element_type=jnp.float32)


def kernel(x_inaux, x_outaux):
    xi = x_inaux.transpose(0, 1, 3, 2)
    xo = x_outaux.transpose(0, 1, 2, 4, 3)

    def xi_map(p, b, t, r):
        return (jnp.where(p == 0, b, 0), jnp.where(p == 0, t * 2 + r, 0), 0, 0)

    def xo_map(p, b, t, r):
        return (jnp.where(p == 0, 0, b), jnp.where(p == 0, 0, t),
                jnp.where(p == 0, 0, r), 0, 0)

    out_t = pl.pallas_call(
        _body,
        grid=(2, 4, 2, 224 // _BR),
        in_specs=[
            pl.BlockSpec((1, 56, _C, _W), xi_map),
            pl.BlockSpec((1, 1, _BR, _C, _W), xo_map),
        ],
        out_specs=pl.BlockSpec((1, 1, _BR, _C, _W), xo_map),
        out_shape=jax.ShapeDtypeStruct(xo.shape, jnp.float32),
        scratch_shapes=[pltpu.VMEM((_C, _W), jnp.float32),
                        pltpu.VMEM((_C, _C), jnp.float32)],
        compiler_params=pltpu.CompilerParams(
            dimension_semantics=("arbitrary", "arbitrary",
                                 "arbitrary", "arbitrary")),
    )(xi, xo)

    return out_t.transpose(0, 1, 2, 4, 3)
